# baseline (device time: 93172 ns/iter reference)
import os

import jax
import jax.numpy as jnp
from jax import lax
from jax.experimental import pallas as pl
from jax.experimental.pallas import tpu as pltpu

N_DEV = 8
N_SUB = 2

_ABLATE = os.environ.get("ABLATE", "")


def kernel(x, w_mat, scale_x, scale_w):
    m_tot, k_per = x.shape
    _, n = w_mat.shape
    m_per = m_tot // N_DEV
    nq = n // (2 * N_SUB)

    def body(x_ref, w_ref, sx_ref, sw_ref, out_ref,
             comm_p, comm_m, send_p, recv_p, send_m, recv_m):
        my = lax.axis_index("i")
        left = (my - 1) % N_DEV
        right = (my + 1) % N_DEV

        def col0(ring, b):
            return (0 if ring == "p" else N_SUB * nq) + b * nq

        nh = N_SUB * nq
        w_bf = {"p": w_ref[:, :nh].astype(jnp.bfloat16),
                "m": w_ref[:, nh:].astype(jnp.bfloat16)}

        def contribs(h):
            c_of = {"p": (my - 2 - h) % N_DEV, "m": (my + 2 + h) % N_DEV}
            out = {}
            for ring in ("p", "m"):
                xc = x_ref[pl.ds(c_of[ring] * m_per, m_per), :].astype(
                    jnp.bfloat16)
                full = jax.lax.dot(
                    xc, w_bf[ring], preferred_element_type=jnp.float32)
                for b in range(N_SUB):
                    out[(ring, b)] = full[:, b * nq:(b + 1) * nq].astype(
                        jnp.bfloat16)
            return out

        def buf(ring):
            return comm_p if ring == "p" else comm_m

        if not _ABLATE.startswith("comm"):
            seed = {"p": (my - 1) % N_DEV, "m": (my + 1) % N_DEV}
            for ring in ("p", "m"):
                xc = x_ref[pl.ds(seed[ring] * m_per, m_per), :].astype(
                    jnp.bfloat16)
                full = jax.lax.dot(
                    xc, w_bf[ring], preferred_element_type=jnp.float32)
                for b in range(N_SUB):
                    buf(ring)[0, b] = full[:, b * nq:(b + 1) * nq].astype(
                        jnp.bfloat16)

        barrier_sem = pltpu.get_barrier_semaphore()
        for nbr in (left, right):
            pl.semaphore_signal(
                barrier_sem, inc=1,
                device_id=(nbr,), device_id_type=pl.DeviceIdType.MESH,
            )
        pl.semaphore_wait(barrier_sem, 2)

        def mk(ring, h, b):
            ssem, rsem, tgt = (
                (send_p, recv_p, right) if ring == "p"
                else (send_m, recv_m, left))
            return pltpu.make_async_remote_copy(
                src_ref=buf(ring).at[h, b],
                dst_ref=buf(ring).at[h + 1, b],
                send_sem=ssem.at[h, b],
                recv_sem=rsem.at[h, b],
                device_id=(tgt,),
                device_id_type=pl.DeviceIdType.MESH,
            )

        order = [("p", 0), ("m", 0), ("p", 1), ("m", 1)]
        if _ABLATE == "comm1":
            order = [("p", 0), ("p", 1)]
        rd = {}

        if _ABLATE == "compute":
            a = contribs(0)
            for h in range(N_DEV - 1):
                for idx, (ring, b) in enumerate(order):
                    if h < N_DEV - 2:
                        buf(ring)[h + 1, b] = buf(ring)[h + 1, b] + a[(ring, b)]
                    else:
                        scale = sx_ref[0] * sw_ref[0]
                        acc = (buf(ring)[h + 1, b].astype(jnp.float32)
                               + a[(ring, b)].astype(jnp.float32))
                        c0 = col0(ring, b)
                        out_ref[:, c0:c0 + nq] = jnp.maximum(acc * scale, 0.0)
                    if idx == 1 and h < N_DEV - 2:
                        a = contribs(h + 1)
            return

        for ring, b in order:
            rd[(ring, 0, b)] = mk(ring, 0, b)
            rd[(ring, 0, b)].start()

        if not _ABLATE.startswith("comm"):
            a = contribs(0)

        for h in range(N_DEV - 1):
            a_next = None
            for idx, (ring, b) in enumerate(order):
                rd[(ring, h, b)].wait_recv()
                if h < N_DEV - 2:
                    if not _ABLATE.startswith("comm"):
                        buf(ring)[h + 1, b] = buf(ring)[h + 1, b] + a[(ring, b)]
                    nxt = mk(ring, h + 1, b)
                    rd[(ring, h + 1, b)] = nxt
                    nxt.start()
                elif not _ABLATE.startswith("comm"):
                    scale = sx_ref[0] * sw_ref[0]
                    acc = (buf(ring)[h + 1, b].astype(jnp.float32)
                           + a[(ring, b)].astype(jnp.float32))
                    c0 = col0(ring, b)
                    out_ref[:, c0:c0 + nq] = jnp.maximum(acc * scale, 0.0)
                if idx == 1 and h < N_DEV - 2 and not _ABLATE.startswith("comm"):
                    a_next = contribs(h + 1)
            a = a_next

        if _ABLATE.startswith("comm"):
            out_ref[:, :] = jnp.zeros((m_per, n), jnp.float32)

        for r in rd.values():
            r.wait_send()

    return pl.pallas_call(
        body,
        out_shape=jax.ShapeDtypeStruct((m_per, n), jnp.float32),
        in_specs=[
            pl.BlockSpec(memory_space=pltpu.VMEM),
            pl.BlockSpec(memory_space=pltpu.VMEM),
            pl.BlockSpec(memory_space=pltpu.SMEM),
            pl.BlockSpec(memory_space=pltpu.SMEM),
        ],
        out_specs=pl.BlockSpec(memory_space=pltpu.VMEM),
        scratch_shapes=[
            pltpu.VMEM((N_DEV, N_SUB, m_per, nq), jnp.bfloat16),
            pltpu.VMEM((N_DEV, N_SUB, m_per, nq), jnp.bfloat16),
            pltpu.SemaphoreType.DMA((N_DEV - 1, N_SUB)),
            pltpu.SemaphoreType.DMA((N_DEV - 1, N_SUB)),
            pltpu.SemaphoreType.DMA((N_DEV - 1, N_SUB)),
            pltpu.SemaphoreType.DMA((N_DEV - 1, N_SUB)),
        ],
        compiler_params=pltpu.CompilerParams(collective_id=0),
    )(x, w_mat, scale_x, scale_w)


# device time: 89088 ns/iter; 1.0458x vs baseline; 1.0458x over previous
import jax
import jax.numpy as jnp
from jax import lax
from jax.experimental import pallas as pl
from jax.experimental.pallas import tpu as pltpu

N_DEV = 8

ORDERINGS = (
    (0, 768, (1, 3, 4)),
    (768, 768, (3, 4, 1)),
    (1536, 512, (4, 1, 3)),
)


def _kept_offsets(masks, k):
    rest = masks[k + 1:]
    offs = [0]
    for m in rest:
        offs = offs + [o ^ m for o in offs]
    return sorted(offs)


def kernel(x, w_mat, scale_x, scale_w):
    m_tot, k_per = x.shape
    _, n = w_mat.shape
    m_per = m_tot // N_DEV

    def body(x_ref, w_ref, sx_ref, sw_ref, out_ref,
             p_a, p_b, p_c, r_a, r_b, r_c,
             ssem_a, rsem_a, ssem_b, rsem_b, ssem_c, rsem_c):
        my = lax.axis_index("i")
        pbufs = (p_a, p_b, p_c)
        rbufs = (r_a, r_b, r_c)
        ssems = (ssem_a, ssem_b, ssem_c)
        rsems = (rsem_a, rsem_b, rsem_c)

        w_bf = w_ref[:, :].astype(jnp.bfloat16)

        def compute_chunk(c):
            xc = x_ref[pl.ds(c * m_per, m_per), :].astype(jnp.bfloat16)
            full = jax.lax.dot(xc, w_bf, preferred_element_type=jnp.float32)
            for o, (c0, cw, _) in enumerate(ORDERINGS):
                pbufs[o][c] = full[:, c0:c0 + cw].astype(jnp.bfloat16)

        for off in range(1, N_DEV):
            compute_chunk(my ^ off)

        barrier_sem = pltpu.get_barrier_semaphore()
        for mask in (1, 3, 4):
            pl.semaphore_signal(
                barrier_sem, inc=1,
                device_id=(my ^ mask,), device_id_type=pl.DeviceIdType.MESH,
            )
        pl.semaphore_wait(barrier_sem, 3)

        slot_base = (0, 4, 6)
        pending = []

        def start_phase(o, k):
            c0, cw, masks = ORDERINGS[o]
            mk = masks[k]
            kept = _kept_offsets(masks, k)
            rds = []
            for j, t in enumerate(kept):
                c = my ^ (t ^ mk)
                rdma = pltpu.make_async_remote_copy(
                    src_ref=pbufs[o].at[c],
                    dst_ref=rbufs[o].at[slot_base[k] + j],
                    send_sem=ssems[o].at[slot_base[k] + j],
                    recv_sem=rsems[o].at[slot_base[k] + j],
                    device_id=(my ^ mk,),
                    device_id_type=pl.DeviceIdType.MESH,
                )
                rdma.start()
                pending.append(rdma)
                rds.append((rdma, t))
            return rds

        def finish_phase(o, k, rds):
            _, _, masks = ORDERINGS[o]
            for rdma, t in rds:
                rdma.wait_recv()
                slot = slot_base[k] + [tt for _, tt in rds].index(t)
                c = my ^ t
                if k == 2:
                    c0, cw, _ = ORDERINGS[o]
                    scale = sx_ref[0] * sw_ref[0]
                    acc = (pbufs[o][c].astype(jnp.float32)
                           + rbufs[o][slot].astype(jnp.float32))
                    out_ref[:, c0:c0 + cw] = jnp.maximum(acc * scale, 0.0)
                else:
                    pbufs[o][c] = pbufs[o][c] + rbufs[o][slot]

        rds1 = [start_phase(o, 0) for o in range(3)]
        compute_chunk(my)
        for o in range(3):
            finish_phase(o, 0, rds1[o])

        rds2 = [start_phase(o, 1) for o in range(3)]
        for o in range(3):
            finish_phase(o, 1, rds2[o])

        rds3 = [start_phase(o, 2) for o in range(3)]
        for o in range(3):
            finish_phase(o, 2, rds3[o])

        for rdma in pending:
            rdma.wait_send()

    scratch = []
    for _, cw, _m in ORDERINGS:
        scratch.append(pltpu.VMEM((N_DEV, m_per, cw), jnp.bfloat16))
    for _, cw, _m in ORDERINGS:
        scratch.append(pltpu.VMEM((7, m_per, cw), jnp.bfloat16))
    for _ in range(3):
        scratch.append(pltpu.SemaphoreType.DMA((7,)))
        scratch.append(pltpu.SemaphoreType.DMA((7,)))

    scratch_shapes = scratch[:3] + scratch[3:6] + scratch[6:]

    return pl.pallas_call(
        body,
        out_shape=jax.ShapeDtypeStruct((m_per, n), jnp.float32),
        in_specs=[
            pl.BlockSpec(memory_space=pltpu.VMEM),
            pl.BlockSpec(memory_space=pltpu.VMEM),
            pl.BlockSpec(memory_space=pltpu.SMEM),
            pl.BlockSpec(memory_space=pltpu.SMEM),
        ],
        out_specs=pl.BlockSpec(memory_space=pltpu.VMEM),
        scratch_shapes=scratch_shapes,
        compiler_params=pltpu.CompilerParams(
            collective_id=0, vmem_limit_bytes=100 * 1024 * 1024),
    )(x, w_mat, scale_x, scale_w)


# device time: 76996 ns/iter; 1.2101x vs baseline; 1.1570x over previous
import jax
import jax.numpy as jnp
from jax import lax
from jax.experimental import pallas as pl
from jax.experimental.pallas import tpu as pltpu

N_DEV = 8

ORDERINGS = (
    (0, 768, (1, 3, 4)),
    (768, 768, (3, 4, 1)),
    (1536, 512, (4, 1, 3)),
)


def _kept(masks, k):
    m1, m2, m3 = masks
    if k == 0:
        return [m2, m2 ^ m3, m3, 0]
    if k == 1:
        return [m3, 0]
    return [0]


_SLOT_BASE = (0, 4, 6)


def kernel(x, w_mat, scale_x, scale_w):
    m_tot, k_per = x.shape
    _, n = w_mat.shape
    m_per = m_tot // N_DEV

    def body(x_ref, w_ref, sx_ref, sw_ref, out_ref,
             p_a, p_b, p_c, r_a, r_b, r_c,
             ssem_a, rsem_a, ssem_b, rsem_b, ssem_c, rsem_c):
        my = lax.axis_index("i")
        pbufs = (p_a, p_b, p_c)
        rbufs = (r_a, r_b, r_c)
        ssems = (ssem_a, ssem_b, ssem_c)
        rsems = (rsem_a, rsem_b, rsem_c)

        barrier_sem = pltpu.get_barrier_semaphore()
        for mask in (1, 3, 4):
            pl.semaphore_signal(
                barrier_sem, inc=1,
                device_id=(my ^ mask,), device_id_type=pl.DeviceIdType.MESH,
            )
        pl.semaphore_wait(barrier_sem, 3)

        w_bf = w_ref[:, :].astype(jnp.bfloat16)

        def compute_chunk(o, c):
            c0, cw, _ = ORDERINGS[o]
            xc = x_ref[pl.ds(c * m_per, m_per), :].astype(jnp.bfloat16)
            part = jax.lax.dot(
                xc, w_bf[:, c0:c0 + cw], preferred_element_type=jnp.float32)
            pbufs[o][c] = part.astype(jnp.bfloat16)

        pending = []

        def start_msg(o, k, j):
            masks = ORDERINGS[o][2]
            t = _kept(masks, k)[j]
            slot = _SLOT_BASE[k] + j
            rdma = pltpu.make_async_remote_copy(
                src_ref=pbufs[o].at[my ^ (t ^ masks[k])],
                dst_ref=rbufs[o].at[slot],
                send_sem=ssems[o].at[slot],
                recv_sem=rsems[o].at[slot],
                device_id=(my ^ masks[k],),
                device_id_type=pl.DeviceIdType.MESH,
            )
            rdma.start()
            pending.append(rdma)
            return rdma

        rd = {}

        for o in range(3):
            masks = ORDERINGS[o][2]
            for j, t in enumerate(_kept(masks, 0)):
                compute_chunk(o, my ^ (t ^ masks[0]))
                rd[(o, 0, j)] = start_msg(o, 0, j)

        for o in range(3):
            masks = ORDERINGS[o][2]
            for t in _kept(masks, 0):
                compute_chunk(o, my ^ t)

        def add_msg(o, k, j):
            masks = ORDERINGS[o][2]
            t = _kept(masks, k)[j]
            slot = _SLOT_BASE[k] + j
            rd[(o, k, j)].wait_recv()
            c = my ^ t
            pbufs[o][c] = pbufs[o][c] + rbufs[o][slot]

        for o in range(3):
            add_msg(o, 0, 0)
        for o in range(3):
            add_msg(o, 0, 1)
            rd[(o, 1, 0)] = start_msg(o, 1, 0)
            rd[(o, 1, 1)] = start_msg(o, 1, 1)
        for o in range(3):
            add_msg(o, 0, 2)
        for o in range(3):
            add_msg(o, 1, 0)
            rd[(o, 2, 0)] = start_msg(o, 2, 0)
        for o in range(3):
            add_msg(o, 0, 3)
        for o in range(3):
            add_msg(o, 1, 1)

        scale = sx_ref[0] * sw_ref[0]
        for o in range(3):
            c0, cw, _ = ORDERINGS[o]
            rd[(o, 2, 0)].wait_recv()
            acc = (pbufs[o][my].astype(jnp.float32)
                   + rbufs[o][6].astype(jnp.float32))
            out_ref[:, c0:c0 + cw] = jnp.maximum(acc * scale, 0.0)

        for rdma in pending:
            rdma.wait_send()

    scratch_shapes = (
        [pltpu.VMEM((N_DEV, m_per, cw), jnp.bfloat16) for _, cw, _m in ORDERINGS]
        + [pltpu.VMEM((7, m_per, cw), jnp.bfloat16) for _, cw, _m in ORDERINGS]
    )
    for _ in range(3):
        scratch_shapes.append(pltpu.SemaphoreType.DMA((7,)))
        scratch_shapes.append(pltpu.SemaphoreType.DMA((7,)))

    return pl.pallas_call(
        body,
        out_shape=jax.ShapeDtypeStruct((m_per, n), jnp.float32),
        in_specs=[
            pl.BlockSpec(memory_space=pltpu.VMEM),
            pl.BlockSpec(memory_space=pltpu.VMEM),
            pl.BlockSpec(memory_space=pltpu.SMEM),
            pl.BlockSpec(memory_space=pltpu.SMEM),
        ],
        out_specs=pl.BlockSpec(memory_space=pltpu.VMEM),
        scratch_shapes=scratch_shapes,
        compiler_params=pltpu.CompilerParams(
            collective_id=0, vmem_limit_bytes=100 * 1024 * 1024),
    )(x, w_mat, scale_x, scale_w)
